# baseline (speedup 1.0000x reference)
import jax
import jax.numpy as jnp
from jax import lax
from jax.experimental import pallas as pl
from jax.experimental.pallas import tpu as pltpu
from jax.experimental.pallas import tpu_sc as plsc

N_NODES = 10000
D_FEAT = 128
N_EDGES = 320000

NC = 2
NS = 16
NW = NC * NS
LANES = 16
DBLK = D_FEAT // LANES

K = 112
CH = 92
E_PAD = NW * CH * K

NRB = 3
NEB = 6
UNROLL = 6

WPT = 624
TAIL = N_NODES - NS * WPT

_GATHER_DNUMS = lax.GatherDimensionNumbers(
    offset_dims=(), collapsed_slice_dims=(0,), start_index_map=(0,))


def _bcast_lane(vv, j):
    return lax.gather(vv, jnp.full((LANES, 1), j, jnp.int32), _GATHER_DNUMS,
                      (1,), mode=lax.GatherScatterMode.PROMISE_IN_BOUNDS)


def _sc_body(edata_hbm, evals_hbm, emb_hbm, part_hbm, acc,
             rb0, rb1, rb2, eb0, eb1, eb2, eb3, eb4, eb5,
             ev0, ev1, ev2, ev3, ev4, ev5,
             sg0, sg1, sg2, ss0, ss1, ss2,
             se0, se1, se2, se3, se4, se5):
    cid = lax.axis_index("c")
    sid = lax.axis_index("s")
    wid = cid * NS + sid

    rbs = (rb0, rb1, rb2)
    ebs = (eb0, eb1, eb2, eb3, eb4, eb5)
    evs = (ev0, ev1, ev2, ev3, ev4, ev5)
    sgs = (sg0, sg1, sg2)
    sss = (ss0, ss1, ss2)
    ses = (se0, se1, se2, se3, se4, se5)

    def prefetch_edges(i, t):
        ic = lax.rem(i, jnp.int32(CH))
        pltpu.async_copy(edata_hbm.at[wid, ic], ebs[t % NEB], ses[t % NEB])
        pltpu.async_copy(evals_hbm.at[wid, ic], evs[t % NEB], ses[t % NEB])

    def wait_edges(t):
        pltpu.make_async_copy(edata_hbm.at[wid, 0], ebs[t % NEB],
                              ses[t % NEB]).wait()
        pltpu.make_async_copy(evals_hbm.at[wid, 0], evs[t % NEB],
                              ses[t % NEB]).wait()

    def issue_gather(i, t):
        wait_edges(t)
        pltpu.async_copy(emb_hbm.at[ebs[t % NEB].at[1]], rbs[t % NRB],
                         sgs[t % NRB])

    def wait_gather(t):
        pltpu.make_async_copy(emb_hbm.at[pl.ds(0, K)], rbs[t % NRB],
                              sgs[t % NRB]).wait()

    def issue_scatter(t):
        pltpu.async_copy(rbs[t % NRB], acc.at[ebs[t % NEB].at[0]],
                         sss[t % NRB], add=True)

    def wait_scatter(t):
        pltpu.make_async_copy(rbs[t % NRB], acc.at[pl.ds(0, K)],
                              sss[t % NRB]).wait()

    def scale(t):
        rb, ev = rbs[t % NRB], evs[t % NEB]

        def blk(b, carry):
            vv = ev[pl.ds(b * LANES, LANES)]
            for j in range(LANES):
                e = b * LANES + j
                bv = _bcast_lane(vv, j)
                for d in range(DBLK):
                    sl = pl.ds(d * LANES, LANES)
                    rb[e, sl] = rb[e, sl] * bv
            return carry

        lax.fori_loop(0, K // LANES, blk, 0)

    def step(i, t, first):
        if not first:
            wait_scatter(t + 1)
        issue_gather(i + 1, t + 1)
        wait_gather(t)
        scale(t)
        issue_scatter(t)
        prefetch_edges(i + 2, t + 2)

    prefetch_edges(jnp.int32(0), 0)
    prefetch_edges(jnp.int32(1), 1)

    def zrow(r, carry):
        for d in range(DBLK):
            rb0[r, pl.ds(d * LANES, LANES)] = jnp.zeros((LANES,), jnp.float32)
        return carry

    lax.fori_loop(0, K, zrow, 0)
    rbase = sid * WPT
    off = 0
    while off < WPT:
        zn = min(K, WPT - off)
        pltpu.sync_copy(rb0.at[pl.ds(0, zn)], acc.at[pl.ds(rbase + off, zn)])
        off += zn

    @pl.when(sid == NS - 1)
    def _zero_tail():
        pltpu.sync_copy(rb0.at[pl.ds(0, TAIL)], acc.at[pl.ds(NS * WPT, TAIL)])

    plsc.subcore_barrier()

    issue_gather(jnp.int32(0), 0)
    step(jnp.int32(0), 0, True)
    step(jnp.int32(1), 1, True)

    def body(h, carry):
        i0 = 2 + UNROLL * h
        for t in range(UNROLL):
            step(i0 + t, 2 + t, False)
        return carry

    lax.fori_loop(0, (CH - 2) // UNROLL, body, 0)

    wait_edges(CH + 1)
    wait_gather(CH)
    wait_scatter(CH - 2)
    wait_scatter(CH - 1)

    plsc.subcore_barrier()

    pltpu.sync_copy(acc.at[pl.ds(rbase, WPT)],
                    part_hbm.at[cid, pl.ds(rbase, WPT)])

    @pl.when(sid == NS - 1)
    def _write_tail():
        pltpu.sync_copy(acc.at[pl.ds(NS * WPT, TAIL)],
                        part_hbm.at[cid, pl.ds(NS * WPT, TAIL)])


def _sc_partials(edata, evals, embeddings):
    mesh = plsc.VectorSubcoreMesh(core_axis_name="c", subcore_axis_name="s",
                                  num_cores=NC, num_subcores=NS)
    f = pl.kernel(
        _sc_body,
        out_type=jax.ShapeDtypeStruct((NC, N_NODES, D_FEAT), jnp.float32),
        mesh=mesh,
        scratch_types=(
            [pltpu.VMEM_SHARED((N_NODES, D_FEAT), jnp.float32)]
            + [pltpu.VMEM((K, D_FEAT), jnp.float32)] * NRB
            + [pltpu.VMEM((2, K), jnp.int32)] * NEB
            + [pltpu.VMEM((K,), jnp.float32)] * NEB
            + [pltpu.SemaphoreType.DMA] * (2 * NRB + NEB)
        ),
    )
    return f(edata, evals, embeddings)


TCB = 1000


def _tc_body(emb_ref, p0_ref, p1_ref, w_ref, b_ref, o_ref):
    x = emb_ref[...] + p0_ref[...] + p1_ref[...]
    h = lax.dot_general(x, w_ref[...], (((1,), (1,)), ((), ())),
                        preferred_element_type=jnp.float32)
    h = h + b_ref[...]
    o_ref[...] = jnp.where(h >= 0, h, 0.01 * h)


def _tc_finish(embeddings, p0, p1, W, b2):
    blk = lambda i: (i, 0)
    return pl.pallas_call(
        _tc_body,
        grid=(N_NODES // TCB,),
        in_specs=[
            pl.BlockSpec((TCB, D_FEAT), blk),
            pl.BlockSpec((TCB, D_FEAT), blk),
            pl.BlockSpec((TCB, D_FEAT), blk),
            pl.BlockSpec((D_FEAT, D_FEAT), lambda i: (0, 0)),
            pl.BlockSpec((1, D_FEAT), lambda i: (0, 0)),
        ],
        out_specs=pl.BlockSpec((TCB, D_FEAT), blk),
        out_shape=jax.ShapeDtypeStruct((N_NODES, D_FEAT), jnp.float32),
    )(embeddings, p0, p1, W, b2)


def kernel(edge_index, edge_values, embeddings, W, b):
    rows = edge_index[0].astype(jnp.int32)
    cols = edge_index[1].astype(jnp.int32)
    vals = edge_values.astype(jnp.float32)
    emb = embeddings.astype(jnp.float32)

    pad = E_PAD - N_EDGES
    z = jnp.zeros((pad,), jnp.int32)
    rows_p = jnp.concatenate([rows, z]).reshape(NW, CH, K)
    cols_p = jnp.concatenate([cols, z]).reshape(NW, CH, K)
    edata = jnp.stack([rows_p, cols_p], axis=2)
    evals = jnp.concatenate([vals, jnp.zeros((pad,), jnp.float32)])
    evals = evals.reshape(NW, CH, K)

    part = _sc_partials(edata, evals, emb)
    return _tc_finish(emb, part[0], part[1], W.astype(jnp.float32),
                      b.astype(jnp.float32).reshape(1, D_FEAT))


# speedup vs baseline: 3.4153x; 3.4153x over previous
import jax
import jax.numpy as jnp
from jax import lax
from jax.experimental import pallas as pl
from jax.experimental.pallas import tpu as pltpu
from jax.experimental.pallas import tpu_sc as plsc

N_NODES = 10000
D_FEAT = 128
N_EDGES = 320000

NC = 2
NS = 16
NW = NC * NS
LANES = 16
DBLK = D_FEAT // LANES

K = 112
CH = 92
E_PAD = NW * CH * K

NRB = 3
NEB = 6
UNROLL = 6

WPT = 624
TAIL = N_NODES - NS * WPT

_GATHER_DNUMS = lax.GatherDimensionNumbers(
    offset_dims=(), collapsed_slice_dims=(0,), start_index_map=(0,))


def _bcast_lane(vv, j):
    return lax.gather(vv, jnp.full((LANES, 1), j, jnp.int32), _GATHER_DNUMS,
                      (1,), mode=lax.GatherScatterMode.PROMISE_IN_BOUNDS)


def _sc_body(edata_hbm, evals_hbm, emb_hbm, part_hbm, acc,
             rb0, rb1, rb2, eb0, eb1, eb2, eb3, eb4, eb5,
             ev0, ev1, ev2, ev3, ev4, ev5,
             sg0, sg1, sg2, ss0, ss1, ss2,
             se0, se1, se2, se3, se4, se5):
    cid = lax.axis_index("c")
    sid = lax.axis_index("s")
    wid = cid * NS + sid

    rbs = (rb0, rb1, rb2)
    ebs = (eb0, eb1, eb2, eb3, eb4, eb5)
    evs = (ev0, ev1, ev2, ev3, ev4, ev5)
    sgs = (sg0, sg1, sg2)
    sss = (ss0, ss1, ss2)
    ses = (se0, se1, se2, se3, se4, se5)

    def prefetch_edges(i, t):
        ic = lax.rem(i, jnp.int32(CH))
        pltpu.async_copy(edata_hbm.at[wid, ic], ebs[t % NEB], ses[t % NEB])
        pltpu.async_copy(evals_hbm.at[wid, ic], evs[t % NEB], ses[t % NEB])

    def wait_edges(t):
        pltpu.make_async_copy(edata_hbm.at[wid, 0], ebs[t % NEB],
                              ses[t % NEB]).wait()
        pltpu.make_async_copy(evals_hbm.at[wid, 0], evs[t % NEB],
                              ses[t % NEB]).wait()

    def issue_gather(i, t):
        wait_edges(t)
        pltpu.async_copy(emb_hbm.at[ebs[t % NEB].at[1]], rbs[t % NRB],
                         sgs[t % NRB])

    def wait_gather(t):
        pltpu.make_async_copy(emb_hbm.at[pl.ds(0, K)], rbs[t % NRB],
                              sgs[t % NRB]).wait()

    def issue_scatter(t):
        pltpu.async_copy(rbs[t % NRB], acc.at[ebs[t % NEB].at[0]],
                         sss[t % NRB], add=True)

    def wait_scatter(t):
        pltpu.make_async_copy(rbs[t % NRB], acc.at[pl.ds(0, K)],
                              sss[t % NRB]).wait()

    def scale(t):
        rb, ev = rbs[t % NRB], evs[t % NEB]

        def blk(b, carry):
            vv = ev[pl.ds(b * LANES, LANES)]
            for j in range(LANES):
                e = b * LANES + j
                bv = _bcast_lane(vv, j)
                for d in range(DBLK):
                    sl = pl.ds(d * LANES, LANES)
                    rb[e, sl] = rb[e, sl] * bv
            return carry

        lax.fori_loop(0, K // LANES, blk, 0)

    def step(i, t, first):
        if not first:
            wait_scatter(t + 1)
        issue_gather(i + 1, t + 1)
        wait_gather(t)
        scale(t)
        issue_scatter(t)
        prefetch_edges(i + 2, t + 2)

    prefetch_edges(jnp.int32(0), 0)
    prefetch_edges(jnp.int32(1), 1)

    def zrow(r, carry):
        for d in range(DBLK):
            rb0[r, pl.ds(d * LANES, LANES)] = jnp.zeros((LANES,), jnp.float32)
        return carry

    lax.fori_loop(0, K, zrow, 0)
    rbase = sid * WPT
    off = 0
    while off < WPT:
        zn = min(K, WPT - off)
        pltpu.sync_copy(rb0.at[pl.ds(0, zn)], acc.at[pl.ds(rbase + off, zn)])
        off += zn

    @pl.when(sid == NS - 1)
    def _zero_tail():
        pltpu.sync_copy(rb0.at[pl.ds(0, TAIL)], acc.at[pl.ds(NS * WPT, TAIL)])

    plsc.subcore_barrier()

    issue_gather(jnp.int32(0), 0)
    step(jnp.int32(0), 0, True)
    step(jnp.int32(1), 1, True)

    def body(h, carry):
        i0 = 2 + UNROLL * h
        for t in range(UNROLL):
            step(i0 + t, 2 + t, False)
        return carry

    lax.fori_loop(0, (CH - 2) // UNROLL, body, 0)

    wait_edges(CH + 1)
    wait_gather(CH)
    wait_scatter(CH - 2)
    wait_scatter(CH - 1)

    plsc.subcore_barrier()

    pltpu.sync_copy(acc.at[pl.ds(rbase, WPT)],
                    part_hbm.at[cid, pl.ds(rbase, WPT)])

    @pl.when(sid == NS - 1)
    def _write_tail():
        pltpu.sync_copy(acc.at[pl.ds(NS * WPT, TAIL)],
                        part_hbm.at[cid, pl.ds(NS * WPT, TAIL)])


def _sc_partials(edata, evals, embeddings):
    mesh = plsc.VectorSubcoreMesh(core_axis_name="c", subcore_axis_name="s",
                                  num_cores=NC, num_subcores=NS)
    f = pl.kernel(
        _sc_body,
        out_type=jax.ShapeDtypeStruct((NC, N_NODES, D_FEAT), jnp.float32),
        mesh=mesh,
        scratch_types=(
            [pltpu.VMEM_SHARED((N_NODES, D_FEAT), jnp.float32)]
            + [pltpu.VMEM((K, D_FEAT), jnp.float32)] * NRB
            + [pltpu.VMEM((2, K), jnp.int32)] * NEB
            + [pltpu.VMEM((K,), jnp.float32)] * NEB
            + [pltpu.SemaphoreType.DMA] * (2 * NRB + NEB)
        ),
    )
    return f(edata, evals, embeddings)


TCB = 1000


def _tc_body(emb_ref, p0_ref, p1_ref, w_ref, b_ref, o_ref):
    x = emb_ref[...] + p0_ref[...] + p1_ref[...]
    h = lax.dot_general(x, w_ref[...], (((1,), (1,)), ((), ())),
                        preferred_element_type=jnp.float32)
    h = h + b_ref[...]
    o_ref[...] = jnp.where(h >= 0, h, 0.01 * h)


def _tc_finish(embeddings, p0, p1, W, b2):
    blk = lambda i: (i, 0)
    return pl.pallas_call(
        _tc_body,
        grid=(N_NODES // TCB,),
        in_specs=[
            pl.BlockSpec((TCB, D_FEAT), blk),
            pl.BlockSpec((TCB, D_FEAT), blk),
            pl.BlockSpec((TCB, D_FEAT), blk),
            pl.BlockSpec((D_FEAT, D_FEAT), lambda i: (0, 0)),
            pl.BlockSpec((1, D_FEAT), lambda i: (0, 0)),
        ],
        out_specs=pl.BlockSpec((TCB, D_FEAT), blk),
        out_shape=jax.ShapeDtypeStruct((N_NODES, D_FEAT), jnp.float32),
    )(embeddings, p0, p1, W, b2)


def kernel(edge_index, edge_values, embeddings, W, b):
    rows = edge_index[0].astype(jnp.int32)
    cols = edge_index[1].astype(jnp.int32)
    vals = edge_values.astype(jnp.float32)
    emb = embeddings.astype(jnp.float32)

    pad = E_PAD - N_EDGES
    z = jnp.arange(pad, dtype=jnp.int32) % N_NODES
    rows_p = jnp.concatenate([rows, z]).reshape(NW, CH, K)
    cols_p = jnp.concatenate([cols, z]).reshape(NW, CH, K)
    edata = jnp.stack([rows_p, cols_p], axis=2)
    evals = jnp.concatenate([vals, jnp.zeros((pad,), jnp.float32)])
    evals = evals.reshape(NW, CH, K)

    part = _sc_partials(edata, evals, emb)
    return _tc_finish(emb, part[0], part[1], W.astype(jnp.float32),
                      b.astype(jnp.float32).reshape(1, D_FEAT))


# speedup vs baseline: 3.6418x; 1.0663x over previous
import jax
import jax.numpy as jnp
from jax import lax
from jax.experimental import pallas as pl
from jax.experimental.pallas import tpu as pltpu
from jax.experimental.pallas import tpu_sc as plsc

N_NODES = 10000
D_FEAT = 128
N_EDGES = 320000

NC = 2
NS = 16
NW = NC * NS
LANES = 16
DBLK = D_FEAT // LANES

K = 80
CH = 125
EPW = CH * K
PRO = 5

NRB = 3
NEB = 6
UNROLL = 6

WPT = 624
TAIL = N_NODES - NS * WPT

_GATHER_DNUMS = lax.GatherDimensionNumbers(
    offset_dims=(), collapsed_slice_dims=(0,), start_index_map=(0,))


def _bcast_lane(vv, j):
    return lax.gather(vv, jnp.full((LANES, 1), j, jnp.int32), _GATHER_DNUMS,
                      (1,), mode=lax.GatherScatterMode.PROMISE_IN_BOUNDS)


def _sc_body(ei_hbm, ev_hbm, emb_hbm, part_hbm, acc,
             rb0, rb1, rb2, er0, er1, er2, er3, er4, er5,
             ec0, ec1, ec2, ec3, ec4, ec5,
             ev0, ev1, ev2, ev3, ev4, ev5,
             sg0, sg1, sg2, ss0, ss1, ss2,
             se0, se1, se2, se3, se4, se5):
    cid = lax.axis_index("c")
    sid = lax.axis_index("s")
    wid = cid * NS + sid
    ebase = wid * EPW

    rbs = (rb0, rb1, rb2)
    ers = (er0, er1, er2, er3, er4, er5)
    ecs = (ec0, ec1, ec2, ec3, ec4, ec5)
    evs = (ev0, ev1, ev2, ev3, ev4, ev5)
    sgs = (sg0, sg1, sg2)
    sss = (ss0, ss1, ss2)
    ses = (se0, se1, se2, se3, se4, se5)

    def prefetch_edges(i, t):
        base = ebase + lax.rem(i, jnp.int32(CH)) * K
        pltpu.async_copy(ei_hbm.at[pl.ds(base, K)], ers[t % NEB],
                         ses[t % NEB])
        pltpu.async_copy(ei_hbm.at[pl.ds(N_EDGES + base, K)], ecs[t % NEB],
                         ses[t % NEB])
        pltpu.async_copy(ev_hbm.at[pl.ds(base, K)], evs[t % NEB],
                         ses[t % NEB])

    def wait_edges(t):
        pltpu.make_async_copy(ei_hbm.at[pl.ds(0, K)], ers[t % NEB],
                              ses[t % NEB]).wait()
        pltpu.make_async_copy(ei_hbm.at[pl.ds(0, K)], ecs[t % NEB],
                              ses[t % NEB]).wait()
        pltpu.make_async_copy(ev_hbm.at[pl.ds(0, K)], evs[t % NEB],
                              ses[t % NEB]).wait()

    def issue_gather(i, t):
        wait_edges(t)
        pltpu.async_copy(emb_hbm.at[ecs[t % NEB]], rbs[t % NRB],
                         sgs[t % NRB])

    def wait_gather(t):
        pltpu.make_async_copy(emb_hbm.at[pl.ds(0, K)], rbs[t % NRB],
                              sgs[t % NRB]).wait()

    def issue_scatter(t):
        pltpu.async_copy(rbs[t % NRB], acc.at[ers[t % NEB]],
                         sss[t % NRB], add=True)

    def wait_scatter(t):
        pltpu.make_async_copy(rbs[t % NRB], acc.at[pl.ds(0, K)],
                              sss[t % NRB]).wait()

    def scale(t):
        rb, ev = rbs[t % NRB], evs[t % NEB]

        def blk(b, carry):
            vv = ev[pl.ds(b * LANES, LANES)]
            for j in range(LANES):
                e = b * LANES + j
                bv = _bcast_lane(vv, j)
                for d in range(DBLK):
                    sl = pl.ds(d * LANES, LANES)
                    rb[e, sl] = rb[e, sl] * bv
            return carry

        lax.fori_loop(0, K // LANES, blk, 0)

    def step(i, t, first):
        if not first:
            wait_scatter(t + 1)
        issue_gather(i + 1, t + 1)
        wait_gather(t)
        scale(t)
        issue_scatter(t)
        prefetch_edges(i + 2, t + 2)

    prefetch_edges(jnp.int32(0), 0)
    prefetch_edges(jnp.int32(1), 1)

    def zrow(r, carry):
        for d in range(DBLK):
            rb0[r, pl.ds(d * LANES, LANES)] = jnp.zeros((LANES,), jnp.float32)
        return carry

    lax.fori_loop(0, K, zrow, 0)
    rbase = sid * WPT
    off = 0
    while off < WPT:
        zn = min(K, WPT - off)
        pltpu.sync_copy(rb0.at[pl.ds(0, zn)], acc.at[pl.ds(rbase + off, zn)])
        off += zn

    @pl.when(sid == NS - 1)
    def _zero_tail():
        pltpu.sync_copy(rb0.at[pl.ds(0, TAIL)], acc.at[pl.ds(NS * WPT, TAIL)])

    plsc.subcore_barrier()

    issue_gather(jnp.int32(0), 0)
    for p in range(PRO):
        step(jnp.int32(p), p, p < 2)

    def body(h, carry):
        i0 = PRO + UNROLL * h
        for t in range(UNROLL):
            step(i0 + t, PRO + t, False)
        return carry

    lax.fori_loop(0, (CH - PRO) // UNROLL, body, 0)

    wait_edges(CH + 1)
    wait_gather(CH)
    wait_scatter(CH - 2)
    wait_scatter(CH - 1)

    plsc.subcore_barrier()

    pltpu.sync_copy(acc.at[pl.ds(rbase, WPT)],
                    part_hbm.at[cid, pl.ds(rbase, WPT)])

    @pl.when(sid == NS - 1)
    def _write_tail():
        pltpu.sync_copy(acc.at[pl.ds(NS * WPT, TAIL)],
                        part_hbm.at[cid, pl.ds(NS * WPT, TAIL)])


def _sc_partials(edge_index, edge_values, embeddings):
    mesh = plsc.VectorSubcoreMesh(core_axis_name="c", subcore_axis_name="s",
                                  num_cores=NC, num_subcores=NS)
    f = pl.kernel(
        _sc_body,
        out_type=jax.ShapeDtypeStruct((NC, N_NODES, D_FEAT), jnp.float32),
        mesh=mesh,
        scratch_types=(
            [pltpu.VMEM_SHARED((N_NODES, D_FEAT), jnp.float32)]
            + [pltpu.VMEM((K, D_FEAT), jnp.float32)] * NRB
            + [pltpu.VMEM((K,), jnp.int32)] * NEB
            + [pltpu.VMEM((K,), jnp.int32)] * NEB
            + [pltpu.VMEM((K,), jnp.float32)] * NEB
            + [pltpu.SemaphoreType.DMA] * (2 * NRB + NEB)
        ),
    )
    return f(edge_index, edge_values, embeddings)


TCB = 1000


def _tc_body(emb_ref, p0_ref, p1_ref, w_ref, b_ref, o_ref):
    x = emb_ref[...] + p0_ref[...] + p1_ref[...]
    h = lax.dot_general(x, w_ref[...], (((1,), (1,)), ((), ())),
                        preferred_element_type=jnp.float32)
    h = h + b_ref[...]
    o_ref[...] = jnp.where(h >= 0, h, 0.01 * h)


def _tc_finish(embeddings, p0, p1, W, b2):
    blk = lambda i: (i, 0)
    return pl.pallas_call(
        _tc_body,
        grid=(N_NODES // TCB,),
        in_specs=[
            pl.BlockSpec((TCB, D_FEAT), blk),
            pl.BlockSpec((TCB, D_FEAT), blk),
            pl.BlockSpec((TCB, D_FEAT), blk),
            pl.BlockSpec((D_FEAT, D_FEAT), lambda i: (0, 0)),
            pl.BlockSpec((1, D_FEAT), lambda i: (0, 0)),
        ],
        out_specs=pl.BlockSpec((TCB, D_FEAT), blk),
        out_shape=jax.ShapeDtypeStruct((N_NODES, D_FEAT), jnp.float32),
    )(embeddings, p0, p1, W, b2)


def kernel(edge_index, edge_values, embeddings, W, b):
    ei = edge_index.astype(jnp.int32).reshape(2 * N_EDGES)
    vals = edge_values.astype(jnp.float32)
    emb = embeddings.astype(jnp.float32)

    part = _sc_partials(ei, vals, emb)
    return _tc_finish(emb, part[0], part[1], W.astype(jnp.float32),
                      b.astype(jnp.float32).reshape(1, D_FEAT))


# speedup vs baseline: 4.1821x; 1.1483x over previous
import jax
import jax.numpy as jnp
from jax import lax
from jax.experimental import pallas as pl
from jax.experimental.pallas import tpu as pltpu
from jax.experimental.pallas import tpu_sc as plsc

N_NODES = 10000
D_FEAT = 128
N_EDGES = 320000

NC = 2
NS = 16
NW = NC * NS
LANES = 16
DBLK = D_FEAT // LANES

K = 80
CH = 125
EPW = CH * K
PRO = 5

NRB = 4
NEB = 8
UNROLL = 8

ZR = 48
WPT = 624
TAIL = N_NODES - NS * WPT

_GATHER_DNUMS = lax.GatherDimensionNumbers(
    offset_dims=(), collapsed_slice_dims=(0,), start_index_map=(0,))


def _bcast_lane(vv, j):
    return lax.gather(vv, jnp.full((LANES, 1), j, jnp.int32), _GATHER_DNUMS,
                      (1,), mode=lax.GatherScatterMode.PROMISE_IN_BOUNDS)


def _sc_body(ei_hbm, ev_hbm, emb_hbm, part_hbm, acc, zbuf,
             rb0, rb1, rb2, rb3, er0, er1, er2, er3, er4, er5, er6, er7,
             ec0, ec1, ec2, ec3, ec4, ec5, ec6, ec7,
             ev0, ev1, ev2, ev3, ev4, ev5, ev6, ev7,
             sg0, sg1, sg2, sg3, ss0, ss1, ss2, ss3,
             se0, se1, se2, se3, se4, se5, se6, se7):
    cid = lax.axis_index("c")
    sid = lax.axis_index("s")
    wid = cid * NS + sid
    ebase = wid * EPW

    rbs = (rb0, rb1, rb2, rb3)
    ers = (er0, er1, er2, er3, er4, er5, er6, er7)
    ecs = (ec0, ec1, ec2, ec3, ec4, ec5, ec6, ec7)
    evs = (ev0, ev1, ev2, ev3, ev4, ev5, ev6, ev7)
    sgs = (sg0, sg1, sg2, sg3)
    sss = (ss0, ss1, ss2, ss3)
    ses = (se0, se1, se2, se3, se4, se5, se6, se7)

    def prefetch_edges(i, t):
        base = ebase + lax.rem(i, jnp.int32(CH)) * K
        pltpu.async_copy(ei_hbm.at[pl.ds(base, K)], ers[t % NEB],
                         ses[t % NEB])
        pltpu.async_copy(ei_hbm.at[pl.ds(N_EDGES + base, K)], ecs[t % NEB],
                         ses[t % NEB])
        pltpu.async_copy(ev_hbm.at[pl.ds(base, K)], evs[t % NEB],
                         ses[t % NEB])

    def wait_edges(t):
        pltpu.make_async_copy(ei_hbm.at[pl.ds(0, K)], ers[t % NEB],
                              ses[t % NEB]).wait()
        pltpu.make_async_copy(ei_hbm.at[pl.ds(0, K)], ecs[t % NEB],
                              ses[t % NEB]).wait()
        pltpu.make_async_copy(ev_hbm.at[pl.ds(0, K)], evs[t % NEB],
                              ses[t % NEB]).wait()

    def issue_gather(i, t):
        wait_edges(t)
        pltpu.async_copy(emb_hbm.at[ecs[t % NEB]], rbs[t % NRB],
                         sgs[t % NRB])

    def wait_gather(t):
        pltpu.make_async_copy(emb_hbm.at[pl.ds(0, K)], rbs[t % NRB],
                              sgs[t % NRB]).wait()

    def issue_scatter(t):
        pltpu.async_copy(rbs[t % NRB], acc.at[ers[t % NEB]],
                         sss[t % NRB], add=True)

    def wait_scatter(t):
        pltpu.make_async_copy(rbs[t % NRB], acc.at[pl.ds(0, K)],
                              sss[t % NRB]).wait()

    def scale(t):
        rb, ev = rbs[t % NRB], evs[t % NEB]

        def blk(b, carry):
            vv = ev[pl.ds(b * LANES, LANES)]
            for j in range(LANES):
                e = b * LANES + j
                bv = _bcast_lane(vv, j)
                for d in range(DBLK):
                    sl = pl.ds(d * LANES, LANES)
                    rb[e, sl] = rb[e, sl] * bv
            return carry

        lax.fori_loop(0, K // LANES, blk, 0)

    def step(i, t, first, prime=False):
        if not first:
            wait_scatter(t + 1)
        if not prime:
            issue_gather(i + 1, t + 1)
        wait_gather(t)
        scale(t)
        issue_scatter(t)
        prefetch_edges(i + 4, t + 4)

    prefetch_edges(jnp.int32(0), 0)
    prefetch_edges(jnp.int32(1), 1)
    issue_gather(jnp.int32(0), 0)
    issue_gather(jnp.int32(1), 1)
    prefetch_edges(jnp.int32(2), 2)
    prefetch_edges(jnp.int32(3), 3)

    def zrow(r, carry):
        for d in range(DBLK):
            zbuf[r, pl.ds(d * LANES, LANES)] = jnp.zeros((LANES,), jnp.float32)
        return carry

    lax.fori_loop(0, ZR, zrow, 0)
    rbase = sid * WPT
    off = 0
    while off < WPT:
        zn = min(ZR, WPT - off)
        pltpu.sync_copy(zbuf.at[pl.ds(0, zn)], acc.at[pl.ds(rbase + off, zn)])
        off += zn

    @pl.when(sid == NS - 1)
    def _zero_tail():
        pltpu.sync_copy(zbuf.at[pl.ds(0, TAIL)], acc.at[pl.ds(NS * WPT, TAIL)])

    plsc.subcore_barrier()

    for p in range(PRO):
        step(jnp.int32(p), p, p < 3, prime=p < 1)

    def body(h, carry):
        i0 = PRO + UNROLL * h
        for t in range(UNROLL):
            step(i0 + t, PRO + t, False)
        return carry

    lax.fori_loop(0, (CH - PRO) // UNROLL, body, 0)

    wait_edges(CH + 1)
    wait_edges(CH + 2)
    wait_edges(CH + 3)
    wait_gather(CH)
    wait_scatter(CH - 3)
    wait_scatter(CH - 2)
    wait_scatter(CH - 1)

    plsc.subcore_barrier()

    pltpu.sync_copy(acc.at[pl.ds(rbase, WPT)],
                    part_hbm.at[cid, pl.ds(rbase, WPT)])

    @pl.when(sid == NS - 1)
    def _write_tail():
        pltpu.sync_copy(acc.at[pl.ds(NS * WPT, TAIL)],
                        part_hbm.at[cid, pl.ds(NS * WPT, TAIL)])


def _sc_partials(edge_index, edge_values, embeddings):
    mesh = plsc.VectorSubcoreMesh(core_axis_name="c", subcore_axis_name="s",
                                  num_cores=NC, num_subcores=NS)
    f = pl.kernel(
        _sc_body,
        out_type=jax.ShapeDtypeStruct((NC, N_NODES, D_FEAT), jnp.float32),
        mesh=mesh,
        scratch_types=(
            [pltpu.VMEM_SHARED((N_NODES, D_FEAT), jnp.float32)]
            + [pltpu.VMEM((ZR, D_FEAT), jnp.float32)]
            + [pltpu.VMEM((K, D_FEAT), jnp.float32)] * NRB
            + [pltpu.VMEM((K,), jnp.int32)] * NEB
            + [pltpu.VMEM((K,), jnp.int32)] * NEB
            + [pltpu.VMEM((K,), jnp.float32)] * NEB
            + [pltpu.SemaphoreType.DMA] * (2 * NRB + NEB)
        ),
    )
    return f(edge_index, edge_values, embeddings)


TCB = 1000


def _tc_body(emb_ref, p0_ref, p1_ref, w_ref, b_ref, o_ref):
    x = emb_ref[...] + p0_ref[...] + p1_ref[...]
    h = lax.dot_general(x, w_ref[...], (((1,), (1,)), ((), ())),
                        preferred_element_type=jnp.float32)
    h = h + b_ref[...]
    o_ref[...] = jnp.where(h >= 0, h, 0.01 * h)


def _tc_finish(embeddings, p0, p1, W, b2):
    blk = lambda i: (i, 0)
    return pl.pallas_call(
        _tc_body,
        grid=(N_NODES // TCB,),
        in_specs=[
            pl.BlockSpec((TCB, D_FEAT), blk),
            pl.BlockSpec((TCB, D_FEAT), blk),
            pl.BlockSpec((TCB, D_FEAT), blk),
            pl.BlockSpec((D_FEAT, D_FEAT), lambda i: (0, 0)),
            pl.BlockSpec((1, D_FEAT), lambda i: (0, 0)),
        ],
        out_specs=pl.BlockSpec((TCB, D_FEAT), blk),
        out_shape=jax.ShapeDtypeStruct((N_NODES, D_FEAT), jnp.float32),
    )(embeddings, p0, p1, W, b2)


def kernel(edge_index, edge_values, embeddings, W, b):
    ei = edge_index.astype(jnp.int32).reshape(2 * N_EDGES)
    vals = edge_values.astype(jnp.float32)
    emb = embeddings.astype(jnp.float32)

    part = _sc_partials(ei, vals, emb)
    return _tc_finish(emb, part[0], part[1], W.astype(jnp.float32),
                      b.astype(jnp.float32).reshape(1, D_FEAT))


# speedup vs baseline: 4.3677x; 1.0444x over previous
import jax
import jax.numpy as jnp
from jax import lax
from jax.experimental import pallas as pl
from jax.experimental.pallas import tpu as pltpu
from jax.experimental.pallas import tpu_sc as plsc

N_NODES = 10000
D_FEAT = 128
N_EDGES = 320000

NC = 2
NS = 16
NW = NC * NS
LANES = 16
DBLK = D_FEAT // LANES

K = 80
CH = 125
EPW = CH * K
PRO = 5

NRB = 4
NEB = 8
UNROLL = 8

ZR = 48
WPT = 624
TAIL = N_NODES - NS * WPT

_GATHER_DNUMS = lax.GatherDimensionNumbers(
    offset_dims=(), collapsed_slice_dims=(0,), start_index_map=(0,))


def _bcast_lane(vv, j):
    return lax.gather(vv, jnp.full((LANES, 1), j, jnp.int32), _GATHER_DNUMS,
                      (1,), mode=lax.GatherScatterMode.PROMISE_IN_BOUNDS)


def _sc_body(ei_hbm, ev_hbm, emb_hbm, part_hbm, acc, zbuf,
             rb0, rb1, rb2, rb3, er0, er1, er2, er3, er4, er5, er6, er7,
             ec0, ec1, ec2, ec3, ec4, ec5, ec6, ec7,
             ev0, ev1, ev2, ev3, ev4, ev5, ev6, ev7,
             sg0, sg1, sg2, sg3, ss0, ss1, ss2, ss3,
             se0, se1, se2, se3, se4, se5, se6, se7):
    cid = lax.axis_index("c")
    sid = lax.axis_index("s")
    wid = cid * NS + sid
    ebase = wid * EPW

    rbs = (rb0, rb1, rb2, rb3)
    ers = (er0, er1, er2, er3, er4, er5, er6, er7)
    ecs = (ec0, ec1, ec2, ec3, ec4, ec5, ec6, ec7)
    evs = (ev0, ev1, ev2, ev3, ev4, ev5, ev6, ev7)
    sgs = (sg0, sg1, sg2, sg3)
    sss = (ss0, ss1, ss2, ss3)
    ses = (se0, se1, se2, se3, se4, se5, se6, se7)

    def prefetch_edges(i, t):
        base = ebase + lax.rem(i, jnp.int32(CH)) * K
        pltpu.async_copy(ei_hbm.at[pl.ds(base, K)], ers[t % NEB],
                         ses[t % NEB])
        pltpu.async_copy(ei_hbm.at[pl.ds(N_EDGES + base, K)], ecs[t % NEB],
                         ses[t % NEB])
        pltpu.async_copy(ev_hbm.at[pl.ds(base, K)], evs[t % NEB],
                         ses[t % NEB])

    def wait_edges(t):
        pltpu.make_async_copy(ei_hbm.at[pl.ds(0, K)], ers[t % NEB],
                              ses[t % NEB]).wait()
        pltpu.make_async_copy(ei_hbm.at[pl.ds(0, K)], ecs[t % NEB],
                              ses[t % NEB]).wait()
        pltpu.make_async_copy(ev_hbm.at[pl.ds(0, K)], evs[t % NEB],
                              ses[t % NEB]).wait()

    def issue_gather(i, t):
        wait_edges(t)
        pltpu.async_copy(emb_hbm.at[ecs[t % NEB]], rbs[t % NRB],
                         sgs[t % NRB])

    def wait_gather(t):
        pltpu.make_async_copy(emb_hbm.at[pl.ds(0, K)], rbs[t % NRB],
                              sgs[t % NRB]).wait()

    def issue_scatter(t):
        pltpu.async_copy(rbs[t % NRB], acc.at[ers[t % NEB]],
                         sss[t % NRB], add=True)

    def wait_scatter(t):
        pltpu.make_async_copy(rbs[t % NRB], acc.at[pl.ds(0, K)],
                              sss[t % NRB]).wait()

    def scale(t):
        rb, ev = rbs[t % NRB], evs[t % NEB]

        def blk(b, carry):
            vv = ev[pl.ds(b * LANES, LANES)]
            for j in range(LANES):
                e = b * LANES + j
                bv = _bcast_lane(vv, j)
                for d in range(DBLK):
                    sl = pl.ds(d * LANES, LANES)
                    rb[e, sl] = rb[e, sl] * bv
            return carry

        lax.fori_loop(0, K // LANES, blk, 0)

    def step(i, t, first, prime=False):
        if not first:
            wait_scatter(t + 2)
        if not prime:
            issue_gather(i + 2, t + 2)
        wait_gather(t)
        scale(t)
        issue_scatter(t)
        prefetch_edges(i + 4, t + 4)

    prefetch_edges(jnp.int32(0), 0)
    prefetch_edges(jnp.int32(1), 1)
    issue_gather(jnp.int32(0), 0)
    issue_gather(jnp.int32(1), 1)
    prefetch_edges(jnp.int32(2), 2)
    prefetch_edges(jnp.int32(3), 3)
    issue_gather(jnp.int32(2), 2)

    def zrow(r, carry):
        for d in range(DBLK):
            zbuf[r, pl.ds(d * LANES, LANES)] = jnp.zeros((LANES,), jnp.float32)
        return carry

    lax.fori_loop(0, ZR, zrow, 0)
    rbase = sid * WPT
    off = 0
    while off < WPT:
        zn = min(ZR, WPT - off)
        pltpu.sync_copy(zbuf.at[pl.ds(0, zn)], acc.at[pl.ds(rbase + off, zn)])
        off += zn

    @pl.when(sid == NS - 1)
    def _zero_tail():
        pltpu.sync_copy(zbuf.at[pl.ds(0, TAIL)], acc.at[pl.ds(NS * WPT, TAIL)])

    plsc.subcore_barrier()

    for p in range(PRO):
        step(jnp.int32(p), p, p < 2, prime=p < 1)

    def body(h, carry):
        i0 = PRO + UNROLL * h
        for t in range(UNROLL):
            step(i0 + t, PRO + t, False)
        return carry

    lax.fori_loop(0, (CH - PRO) // UNROLL, body, 0)

    wait_edges(CH + 2)
    wait_edges(CH + 3)
    wait_gather(CH)
    wait_gather(CH + 1)
    wait_scatter(CH - 2)
    wait_scatter(CH - 1)

    plsc.subcore_barrier()

    pltpu.sync_copy(acc.at[pl.ds(rbase, WPT)],
                    part_hbm.at[cid, pl.ds(rbase, WPT)])

    @pl.when(sid == NS - 1)
    def _write_tail():
        pltpu.sync_copy(acc.at[pl.ds(NS * WPT, TAIL)],
                        part_hbm.at[cid, pl.ds(NS * WPT, TAIL)])


def _sc_partials(edge_index, edge_values, embeddings):
    mesh = plsc.VectorSubcoreMesh(core_axis_name="c", subcore_axis_name="s",
                                  num_cores=NC, num_subcores=NS)
    f = pl.kernel(
        _sc_body,
        out_type=jax.ShapeDtypeStruct((NC, N_NODES, D_FEAT), jnp.float32),
        mesh=mesh,
        scratch_types=(
            [pltpu.VMEM_SHARED((N_NODES, D_FEAT), jnp.float32)]
            + [pltpu.VMEM((ZR, D_FEAT), jnp.float32)]
            + [pltpu.VMEM((K, D_FEAT), jnp.float32)] * NRB
            + [pltpu.VMEM((K,), jnp.int32)] * NEB
            + [pltpu.VMEM((K,), jnp.int32)] * NEB
            + [pltpu.VMEM((K,), jnp.float32)] * NEB
            + [pltpu.SemaphoreType.DMA] * (2 * NRB + NEB)
        ),
    )
    return f(edge_index, edge_values, embeddings)


TCB = 1000


def _tc_body(emb_ref, p0_ref, p1_ref, w_ref, b_ref, o_ref):
    x = emb_ref[...] + p0_ref[...] + p1_ref[...]
    h = lax.dot_general(x, w_ref[...], (((1,), (1,)), ((), ())),
                        preferred_element_type=jnp.float32)
    h = h + b_ref[...]
    o_ref[...] = jnp.where(h >= 0, h, 0.01 * h)


def _tc_finish(embeddings, p0, p1, W, b2):
    blk = lambda i: (i, 0)
    return pl.pallas_call(
        _tc_body,
        grid=(N_NODES // TCB,),
        in_specs=[
            pl.BlockSpec((TCB, D_FEAT), blk),
            pl.BlockSpec((TCB, D_FEAT), blk),
            pl.BlockSpec((TCB, D_FEAT), blk),
            pl.BlockSpec((D_FEAT, D_FEAT), lambda i: (0, 0)),
            pl.BlockSpec((1, D_FEAT), lambda i: (0, 0)),
        ],
        out_specs=pl.BlockSpec((TCB, D_FEAT), blk),
        out_shape=jax.ShapeDtypeStruct((N_NODES, D_FEAT), jnp.float32),
    )(embeddings, p0, p1, W, b2)


def kernel(edge_index, edge_values, embeddings, W, b):
    ei = edge_index.astype(jnp.int32).reshape(2 * N_EDGES)
    vals = edge_values.astype(jnp.float32)
    emb = embeddings.astype(jnp.float32)

    part = _sc_partials(ei, vals, emb)
    return _tc_finish(emb, part[0], part[1], W.astype(jnp.float32),
                      b.astype(jnp.float32).reshape(1, D_FEAT))
